# Initial kernel scaffold; baseline (speedup 1.0000x reference)
import jax
import jax.numpy as jnp
from jax.experimental import pallas as pl


def kernel(x, edge_index, W_in, b_in, W_g1, att_src1, att_dst1, b_g1, g1, beta1, W_g2, att_src2, att_dst2, b_g2, g2, beta2, W_p1, b_p1, W_p2, b_p2):
    raise NotImplementedError("write your pallas kernel here")


# speedup vs baseline: 40.5237x; 40.5237x over previous
import functools

import jax
import jax.numpy as jnp
from jax import lax
from jax.experimental import pallas as pl
from jax.experimental.pallas import tpu as pltpu
from jax.experimental.pallas import tpu_sc as plsc

_N = 10000
_D = 128
_HEADS = 4
_OC = 32
_E = 320000
_NP = 10240
_NW = 32
_CB = 128
_K = 79
_EP = _NW * _CB * _K
_BR = 1024

_f32 = jnp.float32
_i32 = jnp.int32


def _proj_body(x_ref, win_ref, bin_ref, wg_ref, asd_ref, h0_ref, hp_ref,
               asad_ref):
    h0 = jnp.dot(x_ref[...], win_ref[...], preferred_element_type=_f32)
    h0 = h0 + bin_ref[...]
    hp = jnp.dot(h0, wg_ref[...], preferred_element_type=_f32)
    h0_ref[...] = h0
    hp_ref[...] = hp
    asad_ref[...] = jnp.dot(hp, asd_ref[...], preferred_element_type=_f32)


def _finalize(acc0, acc1, den0, den1, hp, asad, hprev, g, beta, bg, exp4):
    as4 = asad[:, 0:4]
    ad4 = asad[:, 4:8]
    t = as4 + ad4
    wself = jnp.exp(jnp.maximum(t, 0.2 * t))
    den4 = den0[:, 0:4] + den1[:, 0:4] + wself
    denx = jnp.dot(den4, exp4, preferred_element_type=_f32)
    accx = acc0 + acc1 + jnp.dot(wself, exp4,
                                 preferred_element_type=_f32) * hp
    gat = accx / (denx + 1e-16) + bg
    z = hprev + gat
    mu = jnp.mean(z, axis=-1, keepdims=True)
    var = jnp.mean((z - mu) ** 2, axis=-1, keepdims=True)
    return jnp.maximum((z - mu) * lax.rsqrt(var + 1e-5) * g + beta, 0.0)


def _mid_body(acc0_ref, acc1_ref, den0_ref, den1_ref, hp_ref, asad_ref,
              hprev_ref, g_ref, beta_ref, bg_ref, exp4_ref, wnext_ref,
              asdnext_ref, hn_ref, hpn_ref, asadn_ref):
    hn = _finalize(acc0_ref[...], acc1_ref[...], den0_ref[...], den1_ref[...],
                   hp_ref[...], asad_ref[...], hprev_ref[...], g_ref[...],
                   beta_ref[...], bg_ref[...], exp4_ref[...])
    hpn = jnp.dot(hn, wnext_ref[...], preferred_element_type=_f32)
    hn_ref[...] = hn
    hpn_ref[...] = hpn
    asadn_ref[...] = jnp.dot(hpn, asdnext_ref[...], preferred_element_type=_f32)


def _tail_body(acc0_ref, acc1_ref, den0_ref, den1_ref, hp_ref, asad_ref,
               hprev_ref, g_ref, beta_ref, bg_ref, exp4_ref, wp1_ref, bp1_ref,
               wp2_ref, bp2_ref, y_ref):
    hn = _finalize(acc0_ref[...], acc1_ref[...], den0_ref[...], den1_ref[...],
                   hp_ref[...], asad_ref[...], hprev_ref[...], g_ref[...],
                   beta_ref[...], bg_ref[...], exp4_ref[...])
    p = jnp.dot(hn, wp1_ref[...], preferred_element_type=_f32) + bp1_ref[...]
    p = jnp.maximum(p, 0.0)
    y_ref[...] = jnp.dot(p, wp2_ref[...], preferred_element_type=_f32) \
        + bp2_ref[...]


def _rb(shape):
    return pl.BlockSpec(shape, lambda i: (i,) + (0,) * (len(shape) - 1))


def _fb(shape):
    return pl.BlockSpec(shape, lambda i: (0,) * len(shape))


_GRID = (_NP // _BR,)

_proj_call = pl.pallas_call(
    _proj_body,
    grid=_GRID,
    in_specs=[_rb((_BR, _D)), _fb((_D, _D)), _fb((1, _D)), _fb((_D, _D)),
              _fb((_D, 16))],
    out_specs=[_rb((_BR, _D)), _rb((_BR, _D)), _rb((_BR, 16))],
    out_shape=[jax.ShapeDtypeStruct((_NP, _D), _f32),
               jax.ShapeDtypeStruct((_NP, _D), _f32),
               jax.ShapeDtypeStruct((_NP, 16), _f32)],
)

_mid_call = pl.pallas_call(
    _mid_body,
    grid=_GRID,
    in_specs=[_rb((_BR, _D)), _rb((_BR, _D)), _rb((_BR, 16)), _rb((_BR, 16)),
              _rb((_BR, _D)), _rb((_BR, 16)), _rb((_BR, _D)),
              _fb((1, _D)), _fb((1, _D)), _fb((1, _D)), _fb((4, _D)),
              _fb((_D, _D)), _fb((_D, 16))],
    out_specs=[_rb((_BR, _D)), _rb((_BR, _D)), _rb((_BR, 16))],
    out_shape=[jax.ShapeDtypeStruct((_NP, _D), _f32),
               jax.ShapeDtypeStruct((_NP, _D), _f32),
               jax.ShapeDtypeStruct((_NP, 16), _f32)],
)

_tail_call = pl.pallas_call(
    _tail_body,
    grid=_GRID,
    in_specs=[_rb((_BR, _D)), _rb((_BR, _D)), _rb((_BR, 16)), _rb((_BR, 16)),
              _rb((_BR, _D)), _rb((_BR, 16)), _rb((_BR, _D)),
              _fb((1, _D)), _fb((1, _D)), _fb((1, _D)), _fb((4, _D)),
              _fb((_D, 16)), _fb((1, 16)), _fb((16, _D)), _fb((1, _D))],
    out_specs=[_rb((_BR, _D))],
    out_shape=[jax.ShapeDtypeStruct((_NP, _D), _f32)],
)


_mesh = plsc.VectorSubcoreMesh(core_axis_name="c", subcore_axis_name="s")


@functools.partial(
    pl.kernel,
    out_type=(jax.ShapeDtypeStruct((2, _NP, _D), _f32),
              jax.ShapeDtypeStruct((2, _NP, 16), _f32)),
    mesh=_mesh,
    compiler_params=pltpu.CompilerParams(needs_layout_passes=False,
                                         use_tc_tiling_on_sc=False),
    scratch_types=[
        pltpu.VMEM_SHARED((_NP, _D), _f32),
        pltpu.VMEM_SHARED((_NP, 16), _f32),
        pltpu.VMEM((_CB,), _i32),
        pltpu.VMEM((1, _CB), _i32),
        pltpu.VMEM((_CB, _D), _f32),
        pltpu.VMEM((_CB, 16), _f32),
        pltpu.VMEM((_CB, 16), _f32),
        pltpu.VMEM((_CB, 16), _f32),
        pltpu.SemaphoreType.DMA,
    ],
)
def _edge_call(hp_hbm, asad_hbm, sidx_hbm, didx_hbm, za_hbm, zd_hbm,
               accs_out, dens_out, acc_sh, den_sh, sidx_v, didx_v, rows_v,
               abuf, bbuf, wbuf, sem):
    c = lax.axis_index("c")
    s = lax.axis_index("s")
    wid = s * 2 + c

    rpt = _NP // 16
    base = s * rpt

    def _zinit(i, carry):
        sl = pl.ds(base + i * _CB, _CB)
        pltpu.sync_copy(za_hbm.at[sl], acc_sh.at[sl])
        return carry

    lax.fori_loop(0, rpt // _CB, _zinit, 0)
    pltpu.sync_copy(zd_hbm.at[pl.ds(base, rpt)], den_sh.at[pl.ds(base, rpt)])

    plsc.subcore_barrier()

    lanes = lax.iota(_i32, 16)
    shift_idx = jnp.minimum(lanes + 4, 15)
    h_idx = [jnp.full((16,), h, _i32) for h in range(_HEADS)]

    def chunk(j, carry):
        pltpu.sync_copy(sidx_hbm.at[wid, j], sidx_v)
        pltpu.sync_copy(didx_hbm.at[wid, pl.ds(j, 1)], didx_v)
        ga = pltpu.async_copy(hp_hbm.at[sidx_v], rows_v, sem)
        gb = pltpu.async_copy(asad_hbm.at[sidx_v], abuf, sem)
        gc = pltpu.async_copy(asad_hbm.at[didx_v.at[0]], bbuf, sem)
        ga.wait()
        gb.wait()
        gc.wait()

        def edge(e, carry2):
            row_e = jnp.full((16,), e, _i32)
            va = abuf[e, :]
            vbs = plsc.load_gather(bbuf, [row_e, shift_idx])
            t = va + vbs
            w = jnp.exp(jnp.maximum(t, 0.2 * t))
            wm = jnp.where(lanes < _HEADS, w, 0.0)
            wbuf[e, :] = wm
            scales = [plsc.load_gather(wbuf, [row_e, h_idx[h]])
                      for h in range(_HEADS)]
            for jj in range(_D // 16):
                sl = pl.ds(jj * 16, 16)
                rows_v[e, sl] = rows_v[e, sl] * scales[jj // 2]
            return carry2

        lax.fori_loop(0, _CB, edge, 0)
        pltpu.sync_copy(rows_v, acc_sh.at[didx_v.at[0]], add=True)
        pltpu.sync_copy(wbuf, den_sh.at[didx_v.at[0]], add=True)
        return carry

    lax.fori_loop(0, _K, chunk, 0)
    plsc.subcore_barrier()

    def _zdrain(i, carry):
        sl = pl.ds(base + i * _CB, _CB)
        pltpu.sync_copy(acc_sh.at[sl], accs_out.at[c, sl])
        return carry

    lax.fori_loop(0, rpt // _CB, _zdrain, 0)
    pltpu.sync_copy(den_sh.at[pl.ds(base, rpt)],
                    dens_out.at[c, pl.ds(base, rpt)])


def _make_asd(a_s, a_d):
    bs = jax.scipy.linalg.block_diag(*[a_s[0, h][None, :]
                                       for h in range(_HEADS)])
    bd = jax.scipy.linalg.block_diag(*[a_d[0, h][None, :]
                                       for h in range(_HEADS)])
    return jnp.concatenate([bs.T, bd.T, jnp.zeros((_D, 8), _f32)], axis=1)


def kernel(x, edge_index, W_in, b_in, W_g1, att_src1, att_dst1, b_g1, g1,
           beta1, W_g2, att_src2, att_dst2, b_g2, g2, beta2, W_p1, b_p1,
           W_p2, b_p2):
    x_pad = jnp.zeros((_NP, _D), _f32).at[:_N].set(x)
    src = edge_index[0].astype(_i32)
    dst = edge_index[1].astype(_i32)
    pad = jnp.full((_EP - _E,), _N, _i32)
    sidx = jnp.concatenate([src, pad]).reshape(_NW, _K, _CB)
    didx = jnp.concatenate([dst, pad]).reshape(_NW, _K, _CB)

    asd1 = _make_asd(att_src1, att_dst1)
    asd2 = _make_asd(att_src2, att_dst2)
    exp4 = jax.scipy.linalg.block_diag(
        *[jnp.ones((1, _OC), _f32)] * _HEADS)
    za = jnp.zeros((_NP, _D), _f32)
    zd = jnp.zeros((_NP, 16), _f32)

    b_in2 = b_in[None, :]
    g1_2, beta1_2, bg1_2 = g1[None, :], beta1[None, :], b_g1[None, :]
    g2_2, beta2_2, bg2_2 = g2[None, :], beta2[None, :], b_g2[None, :]
    bp1_2 = b_p1[None, :]
    wp2_pad = jnp.pad(W_p2, ((0, 0), (0, _D - W_p2.shape[1])))
    bp2_pad = jnp.pad(b_p2, (0, _D - b_p2.shape[0]))[None, :]

    h0, hp1, asad1 = _proj_call(x_pad, W_in, b_in2, W_g1, asd1)
    accs1, dens1 = _edge_call(hp1, asad1, sidx, didx, za, zd)
    h1, hp2, asad2 = _mid_call(accs1[0], accs1[1], dens1[0], dens1[1], hp1,
                               asad1, h0, g1_2, beta1_2, bg1_2, exp4, W_g2,
                               asd2)
    accs2, dens2 = _edge_call(hp2, asad2, sidx, didx, za, zd)
    (y,) = _tail_call(accs2[0], accs2[1], dens2[0], dens2[1], hp2, asad2, h1,
                      g2_2, beta2_2, bg2_2, exp4, W_p1, bp1_2, wp2_pad,
                      bp2_pad)
    return y[:_N, :1]


# speedup vs baseline: 63.7305x; 1.5727x over previous
import functools

import jax
import jax.numpy as jnp
from jax import lax
from jax.experimental import pallas as pl
from jax.experimental.pallas import tpu as pltpu
from jax.experimental.pallas import tpu_sc as plsc

_N = 10000
_D = 128
_HEADS = 4
_OC = 32
_E = 320000
_NP = 10112
_NW = 32
_CB = 112
_K = 90
_ZCH = 128
_EP = _NW * _CB * _K
_BR = 1264

_f32 = jnp.float32
_i32 = jnp.int32


def _proj_body(x_ref, win_ref, bin_ref, wg_ref, asd_ref, h0_ref, hp_ref,
               asad_ref):
    h0 = jnp.dot(x_ref[...], win_ref[...], preferred_element_type=_f32)
    h0 = h0 + bin_ref[...]
    hp = jnp.dot(h0, wg_ref[...], preferred_element_type=_f32)
    h0_ref[...] = h0
    hp_ref[...] = hp
    asad_ref[...] = jnp.dot(hp, asd_ref[...], preferred_element_type=_f32)


def _finalize(acc0, acc1, den0, den1, hp, asad, hprev, g, beta, bg, exp4):
    as4 = asad[:, 0:4]
    ad4 = asad[:, 4:8]
    t = as4 + ad4
    wself = jnp.exp(jnp.maximum(t, 0.2 * t))
    den4 = den0[:, 0:4] + den1[:, 0:4] + wself
    denx = jnp.dot(den4, exp4, preferred_element_type=_f32)
    accx = acc0 + acc1 + jnp.dot(wself, exp4,
                                 preferred_element_type=_f32) * hp
    gat = accx / (denx + 1e-16) + bg
    z = hprev + gat
    mu = jnp.mean(z, axis=-1, keepdims=True)
    var = jnp.mean((z - mu) ** 2, axis=-1, keepdims=True)
    return jnp.maximum((z - mu) * lax.rsqrt(var + 1e-5) * g + beta, 0.0)


def _mid_body(acc0_ref, acc1_ref, den0_ref, den1_ref, hp_ref, asad_ref,
              hprev_ref, g_ref, beta_ref, bg_ref, exp4_ref, wnext_ref,
              asdnext_ref, hn_ref, hpn_ref, asadn_ref):
    hn = _finalize(acc0_ref[...], acc1_ref[...], den0_ref[...], den1_ref[...],
                   hp_ref[...], asad_ref[...], hprev_ref[...], g_ref[...],
                   beta_ref[...], bg_ref[...], exp4_ref[...])
    hpn = jnp.dot(hn, wnext_ref[...], preferred_element_type=_f32)
    hn_ref[...] = hn
    hpn_ref[...] = hpn
    asadn_ref[...] = jnp.dot(hpn, asdnext_ref[...], preferred_element_type=_f32)


def _tail_body(acc0_ref, acc1_ref, den0_ref, den1_ref, hp_ref, asad_ref,
               hprev_ref, g_ref, beta_ref, bg_ref, exp4_ref, wp1_ref, bp1_ref,
               wp2_ref, bp2_ref, y_ref):
    hn = _finalize(acc0_ref[...], acc1_ref[...], den0_ref[...], den1_ref[...],
                   hp_ref[...], asad_ref[...], hprev_ref[...], g_ref[...],
                   beta_ref[...], bg_ref[...], exp4_ref[...])
    p = jnp.dot(hn, wp1_ref[...], preferred_element_type=_f32) + bp1_ref[...]
    p = jnp.maximum(p, 0.0)
    y_ref[...] = jnp.dot(p, wp2_ref[...], preferred_element_type=_f32) \
        + bp2_ref[...]


def _rb(shape):
    return pl.BlockSpec(shape, lambda i: (i,) + (0,) * (len(shape) - 1))


def _fb(shape):
    return pl.BlockSpec(shape, lambda i: (0,) * len(shape))


_GRID = (_NP // _BR,)

_proj_call = pl.pallas_call(
    _proj_body,
    grid=_GRID,
    in_specs=[_rb((_BR, _D)), _fb((_D, _D)), _fb((1, _D)), _fb((_D, _D)),
              _fb((_D, 16))],
    out_specs=[_rb((_BR, _D)), _rb((_BR, _D)), _rb((_BR, 16))],
    out_shape=[jax.ShapeDtypeStruct((_NP, _D), _f32),
               jax.ShapeDtypeStruct((_NP, _D), _f32),
               jax.ShapeDtypeStruct((_NP, 16), _f32)],
)

_mid_call = pl.pallas_call(
    _mid_body,
    grid=_GRID,
    in_specs=[_rb((_BR, _D)), _rb((_BR, _D)), _rb((_BR, 16)), _rb((_BR, 16)),
              _rb((_BR, _D)), _rb((_BR, 16)), _rb((_BR, _D)),
              _fb((1, _D)), _fb((1, _D)), _fb((1, _D)), _fb((4, _D)),
              _fb((_D, _D)), _fb((_D, 16))],
    out_specs=[_rb((_BR, _D)), _rb((_BR, _D)), _rb((_BR, 16))],
    out_shape=[jax.ShapeDtypeStruct((_NP, _D), _f32),
               jax.ShapeDtypeStruct((_NP, _D), _f32),
               jax.ShapeDtypeStruct((_NP, 16), _f32)],
)

_tail_call = pl.pallas_call(
    _tail_body,
    grid=_GRID,
    in_specs=[_rb((_BR, _D)), _rb((_BR, _D)), _rb((_BR, 16)), _rb((_BR, 16)),
              _rb((_BR, _D)), _rb((_BR, 16)), _rb((_BR, _D)),
              _fb((1, _D)), _fb((1, _D)), _fb((1, _D)), _fb((4, _D)),
              _fb((_D, 16)), _fb((1, 16)), _fb((16, _D)), _fb((1, _D))],
    out_specs=[_rb((_BR, _D))],
    out_shape=[jax.ShapeDtypeStruct((_NP, _D), _f32)],
)


_mesh = plsc.VectorSubcoreMesh(core_axis_name="c", subcore_axis_name="s")


@functools.partial(
    pl.kernel,
    out_type=(jax.ShapeDtypeStruct((2, _NP, _D), _f32),
              jax.ShapeDtypeStruct((2, _NP, 16), _f32)),
    mesh=_mesh,
    compiler_params=pltpu.CompilerParams(needs_layout_passes=False,
                                         use_tc_tiling_on_sc=False),
    scratch_types=[
        pltpu.VMEM_SHARED((_NP, _D), _f32),
        pltpu.VMEM_SHARED((_NP, 16), _f32),
        pltpu.VMEM((_CB,), _i32),
        pltpu.VMEM((1, _CB), _i32),
        pltpu.VMEM((_CB, _D), _f32),
        pltpu.VMEM((_CB, 16), _f32),
        pltpu.VMEM((_CB, 16), _f32),
        pltpu.VMEM((_CB, 16), _f32),
        pltpu.VMEM((_CB,), _i32),
        pltpu.VMEM((1, _CB), _i32),
        pltpu.VMEM((_CB, _D), _f32),
        pltpu.VMEM((_CB, 16), _f32),
        pltpu.VMEM((_CB, 16), _f32),
        pltpu.VMEM((_CB, 16), _f32),
        pltpu.SemaphoreType.DMA,
        pltpu.SemaphoreType.DMA,
        pltpu.SemaphoreType.DMA,
        pltpu.SemaphoreType.DMA,
    ],
)
def _edge_call(hp_hbm, asad_hbm, sidx_hbm, didx_hbm, za_hbm, zd_hbm,
               accs_out, dens_out, acc_sh, den_sh,
               sidx0, didx0, rows0, abuf0, bbuf0, wbuf0,
               sidx1, didx1, rows1, abuf1, bbuf1, wbuf1,
               sg0, sg1, ss0, ss1):
    c = lax.axis_index("c")
    s = lax.axis_index("s")
    wid = s * 2 + c

    nch = _NP // _ZCH
    rpt = _NP // 16
    base = s * rpt

    def _zinit(i, carry):
        k = s + 16 * i

        @pl.when(k < nch)
        def _():
            sl = pl.ds(k * _ZCH, _ZCH)
            pltpu.sync_copy(za_hbm.at[sl], acc_sh.at[sl])

        return carry

    lax.fori_loop(0, (nch + 15) // 16, _zinit, 0)
    pltpu.sync_copy(zd_hbm.at[pl.ds(base, rpt)], den_sh.at[pl.ds(base, rpt)])

    lanes = lax.iota(_i32, 16)
    h_idx = [jnp.full((16,), h, _i32) for h in range(_HEADS)]
    hd_idx = [jnp.full((16,), 4 + h, _i32) for h in range(_HEADS)]
    zero16 = jnp.zeros((16,), _f32)

    set0 = (sidx0, didx0, rows0, abuf0, bbuf0, wbuf0, sg0, ss0)
    set1 = (sidx1, didx1, rows1, abuf1, bbuf1, wbuf1, sg1, ss1)

    def load_idx(j, bufs):
        pltpu.sync_copy(sidx_hbm.at[wid, j], bufs[0])
        pltpu.sync_copy(didx_hbm.at[wid, pl.ds(j, 1)], bufs[1])

    def fire_gathers(bufs):
        sidx_v, didx_v, rows_v, abuf, bbuf, _, sg, _ = bufs
        pltpu.async_copy(hp_hbm.at[sidx_v], rows_v, sg)
        pltpu.async_copy(asad_hbm.at[sidx_v], abuf, sg)
        pltpu.async_copy(asad_hbm.at[didx_v.at[0]], bbuf, sg)

    def wait_gathers(bufs):
        sidx_v, didx_v, rows_v, abuf, bbuf, _, sg, _ = bufs
        pltpu.make_async_copy(hp_hbm.at[sidx_v], rows_v, sg).wait()
        pltpu.make_async_copy(asad_hbm.at[sidx_v], abuf, sg).wait()
        pltpu.make_async_copy(asad_hbm.at[didx_v.at[0]], bbuf, sg).wait()

    def fire_scatters(bufs):
        _, didx_v, rows_v, _, _, wbuf, _, ss = bufs
        pltpu.async_copy(rows_v, acc_sh.at[didx_v.at[0]], ss, add=True)
        pltpu.async_copy(wbuf, den_sh.at[didx_v.at[0]], ss, add=True)

    def wait_scatters(bufs):
        _, didx_v, rows_v, _, _, wbuf, _, ss = bufs
        pltpu.make_async_copy(rows_v, acc_sh.at[didx_v.at[0]], ss).wait()
        pltpu.make_async_copy(wbuf, den_sh.at[didx_v.at[0]], ss).wait()

    def compute(bufs):
        _, _, rows_v, abuf, bbuf, wbuf, _, _ = bufs

        def group(g, carry):
            ev = jnp.full((16,), g * 16, _i32) + lanes
            for h in range(_HEADS):
                as_h = plsc.load_gather(abuf, [ev, h_idx[h]])
                ad_h = plsc.load_gather(bbuf, [ev, hd_idx[h]])
                t = as_h + ad_h
                w_h = jnp.exp(jnp.maximum(t, 0.2 * t))
                plsc.store_scatter(wbuf, [ev, h_idx[h]], w_h)
            return carry

        lax.fori_loop(0, _CB // 16, group, 0)

        def edge(e, carry2):
            row_e = jnp.full((16,), e, _i32)
            scales = [plsc.load_gather(wbuf, [row_e, h_idx[h]])
                      for h in range(_HEADS)]
            for jj in range(_D // 16):
                sl = pl.ds(jj * 16, 16)
                rows_v[e, sl] = rows_v[e, sl] * scales[jj // 2]
            return carry2

        lax.fori_loop(0, _CB, edge, 0)

    def phase(j, cur, nxt):
        wait_gathers(cur)

        @pl.when(j + 1 < _K)
        def _prefetch():
            @pl.when(j >= 1)
            def _():
                wait_scatters(nxt)

            load_idx(j + 1, nxt)
            fire_gathers(nxt)

        compute(cur)
        fire_scatters(cur)

    def _zw(e, carry):
        wbuf0[e, :] = zero16
        wbuf1[e, :] = zero16
        return carry

    lax.fori_loop(0, _CB, _zw, 0)

    load_idx(0, set0)
    fire_gathers(set0)
    plsc.subcore_barrier()

    def body(i, carry):
        phase(2 * i, set0, set1)
        phase(2 * i + 1, set1, set0)
        return carry

    lax.fori_loop(0, _K // 2, body, 0)
    wait_scatters(set0)
    wait_scatters(set1)
    plsc.subcore_barrier()

    def _zdrain(i, carry):
        k = s + 16 * i

        @pl.when(k < nch)
        def _():
            sl = pl.ds(k * _ZCH, _ZCH)
            pltpu.sync_copy(acc_sh.at[sl], accs_out.at[c, sl])

        return carry

    lax.fori_loop(0, (nch + 15) // 16, _zdrain, 0)
    pltpu.sync_copy(den_sh.at[pl.ds(base, rpt)],
                    dens_out.at[c, pl.ds(base, rpt)])


def _make_asd(a_s, a_d):
    bs = jax.scipy.linalg.block_diag(*[a_s[0, h][None, :]
                                       for h in range(_HEADS)])
    bd = jax.scipy.linalg.block_diag(*[a_d[0, h][None, :]
                                       for h in range(_HEADS)])
    return jnp.concatenate([bs.T, bd.T, jnp.zeros((_D, 8), _f32)], axis=1)


def kernel(x, edge_index, W_in, b_in, W_g1, att_src1, att_dst1, b_g1, g1,
           beta1, W_g2, att_src2, att_dst2, b_g2, g2, beta2, W_p1, b_p1,
           W_p2, b_p2):
    x_pad = jnp.zeros((_NP, _D), _f32).at[:_N].set(x)
    src = edge_index[0].astype(_i32)
    dst = edge_index[1].astype(_i32)
    pad = jnp.full((_EP - _E,), _N, _i32)
    sidx = jnp.concatenate([src, pad]).reshape(_NW, _K, _CB)
    didx = jnp.concatenate([dst, pad]).reshape(_NW, _K, _CB)

    asd1 = _make_asd(att_src1, att_dst1)
    asd2 = _make_asd(att_src2, att_dst2)
    exp4 = jax.scipy.linalg.block_diag(
        *[jnp.ones((1, _OC), _f32)] * _HEADS)
    za = jnp.zeros((_NP, _D), _f32)
    zd = jnp.zeros((_NP, 16), _f32)

    b_in2 = b_in[None, :]
    g1_2, beta1_2, bg1_2 = g1[None, :], beta1[None, :], b_g1[None, :]
    g2_2, beta2_2, bg2_2 = g2[None, :], beta2[None, :], b_g2[None, :]
    bp1_2 = b_p1[None, :]
    wp2_pad = jnp.pad(W_p2, ((0, 0), (0, _D - W_p2.shape[1])))
    bp2_pad = jnp.pad(b_p2, (0, _D - b_p2.shape[0]))[None, :]

    h0, hp1, asad1 = _proj_call(x_pad, W_in, b_in2, W_g1, asd1)
    accs1, dens1 = _edge_call(hp1, asad1, sidx, didx, za, zd)
    h1, hp2, asad2 = _mid_call(accs1[0], accs1[1], dens1[0], dens1[1], hp1,
                               asad1, h0, g1_2, beta1_2, bg1_2, exp4, W_g2,
                               asd2)
    accs2, dens2 = _edge_call(hp2, asad2, sidx, didx, za, zd)
    (y,) = _tail_call(accs2[0], accs2[1], dens2[0], dens2[1], hp2, asad2, h1,
                      g2_2, beta2_2, bg2_2, exp4, W_p1, bp1_2, wp2_pad,
                      bp2_pad)
    return y[:_N, :1]
